# baseline (speedup 1.0000x reference)
import functools
import jax
import jax.numpy as jnp
from jax import lax
from jax.experimental import pallas as pl
from jax.experimental.pallas import tpu as pltpu
from jax.experimental.pallas import tpu_sc as plsc

DIM = 64
SCALE = 8.0
LANES = 16

NC = 2
NS = 16
NW = NC * NS

BBLK = 128
NBUF = 2


def _emb_body(seq, ids_hbm, table_hbm, out_hbm, idx_v, g0, g1, t0, t1,
              gs0, gs1, ss0, ss1):
    c = lax.axis_index("c")
    s_ax = lax.axis_index("s")
    wid = s_ax * NC + c

    gbufs, tbufs = (g0, g1), (t0, t1)
    gsems, ssems = (gs0, gs1), (ss0, ss1)

    pltpu.sync_copy(ids_hbm.at[:, pl.ds(wid * BBLK, BBLK)], idx_v)

    for b in range(NBUF):
        pltpu.async_copy(table_hbm.at[idx_v.at[b]], gbufs[b], gsems[b])

    qbase = lax.iota(jnp.int32, LANES)
    ngroups = seq // NBUF

    def group_body(go, carry):
        for b in range(NBUF):
            s = go * NBUF + b
            gbuf, tbuf = gbufs[b], tbufs[b]

            pltpu.make_async_copy(table_hbm.at[idx_v.at[b]], gbuf,
                                  gsems[b]).wait()

            @pl.when(go > 0)
            def _():
                pltpu.make_async_copy(tbuf, out_hbm.at[0, :, 0],
                                      ssems[b]).wait()

            def d_body(d, dc):
                dvec = jnp.full((LANES,), d, jnp.int32)
                for k in range(BBLK // LANES):
                    col = plsc.load_gather(
                        gbuf, [qbase + (k * LANES), dvec])
                    tbuf[d // 8, d % 8, pl.ds(k * LANES, LANES)] = col * SCALE
                return dc

            lax.fori_loop(0, DIM, d_body, 0)

            pltpu.async_copy(tbuf, out_hbm.at[s, :, wid], ssems[b])

            @pl.when(go < ngroups - 1)
            def _():
                pltpu.async_copy(table_hbm.at[idx_v.at[s + NBUF]], gbuf,
                                 gsems[b])

        return carry

    lax.fori_loop(0, ngroups, group_body, 0)

    for b in range(NBUF):
        pltpu.make_async_copy(tbufs[b], out_hbm.at[0, :, 0], ssems[b]).wait()


@functools.partial(jax.jit, static_argnames=("batch", "seq"))
def _embed(ids_t, table, batch, seq):
    mesh = plsc.VectorSubcoreMesh(
        core_axis_name="c", subcore_axis_name="s", num_cores=NC,
        num_subcores=NS)
    out5 = pl.kernel(
        functools.partial(_emb_body, seq),
        out_type=jax.ShapeDtypeStruct(
            (seq, DIM // 8, batch // BBLK, 8, BBLK), jnp.float32),
        mesh=mesh,
        scratch_types=[
            pltpu.VMEM((seq, BBLK), jnp.int32),
            pltpu.VMEM((BBLK, DIM), jnp.float32),
            pltpu.VMEM((BBLK, DIM), jnp.float32),
            pltpu.VMEM((DIM // 8, 8, BBLK), jnp.float32),
            pltpu.VMEM((DIM // 8, 8, BBLK), jnp.float32),
            pltpu.SemaphoreType.DMA,
            pltpu.SemaphoreType.DMA,
            pltpu.SemaphoreType.DMA,
            pltpu.SemaphoreType.DMA,
        ],
        compiler_params=pltpu.CompilerParams(
            use_tc_tiling_on_sc=False, needs_layout_passes=False),
    )(ids_t, table)
    return out5.transpose(2, 4, 0, 1, 3).reshape(batch, seq, DIM)


def kernel(input_ids, table):
    batch, seq = input_ids.shape
    ids_t = input_ids.T.astype(jnp.int32)
    return _embed(ids_t, table, batch, seq)


# speedup vs baseline: 1.2996x; 1.2996x over previous
import functools
import jax
import jax.numpy as jnp
from jax import lax
from jax.experimental import pallas as pl
from jax.experimental.pallas import tpu as pltpu
from jax.experimental.pallas import tpu_sc as plsc

DIM = 64
SCALE = 8.0
LANES = 16

NC = 2
NS = 16
NW = NC * NS

BBLK = 128
ROWPAD = 64
PITCH = 129
NBUF = 2


def _emb_body(seq, ids_hbm, table_hbm, out_hbm, idx_v, g0, g1, f0, f1,
              t0, t1, gs0, gs1, ss0, ss1):
    c = lax.axis_index("c")
    s_ax = lax.axis_index("s")
    wid = s_ax * NC + c

    gbufs, fbufs, tbufs = (g0, g1), (f0, f1), (t0, t1)
    gsems, ssems = (gs0, gs1), (ss0, ss1)

    pltpu.sync_copy(ids_hbm.at[:, pl.ds(wid * BBLK, BBLK)], idx_v)

    for b in range(NBUF):
        pltpu.async_copy(table_hbm.at[idx_v.at[b]], gbufs[b], gsems[b])

    qbase = lax.iota(jnp.int32, LANES)
    qpitch = qbase * PITCH
    ngroups = seq // NBUF

    def group_body(go, carry):
        for b in range(NBUF):
            s = go * NBUF + b
            gbuf, fbuf, tbuf = gbufs[b], fbufs[b], tbufs[b]

            pltpu.make_async_copy(table_hbm.at[idx_v.at[b]], gbuf,
                                  gsems[b]).wait()

            @pl.when(go > 0)
            def _():
                pltpu.make_async_copy(tbuf, out_hbm.at[0, :, 0],
                                      ssems[b]).wait()

            def q_body(q, qc):
                for k in range(DIM // LANES):
                    v = gbuf[q, pl.ds(k * LANES, LANES)]
                    plsc.store_scatter(
                        fbuf, [qpitch + (k * LANES * PITCH + q)], v * SCALE)
                return qc

            lax.fori_loop(0, BBLK, q_body, 0, unroll=4)

            def d_body(d, dc):
                dbase = qbase + d * PITCH
                for m in range(BBLK // LANES):
                    v = plsc.load_gather(fbuf, [dbase + m * LANES])
                    tbuf[d // 8, d % 8, pl.ds(m * LANES, LANES)] = v
                return dc

            lax.fori_loop(0, DIM, d_body, 0, unroll=2)

            pltpu.async_copy(tbuf, out_hbm.at[s, :, wid], ssems[b])

            @pl.when(go < ngroups - 1)
            def _():
                pltpu.async_copy(table_hbm.at[idx_v.at[s + NBUF]], gbuf,
                                 gsems[b])

        return carry

    lax.fori_loop(0, ngroups, group_body, 0)

    for b in range(NBUF):
        pltpu.make_async_copy(tbufs[b], out_hbm.at[0, :, 0], ssems[b]).wait()


@functools.partial(jax.jit, static_argnames=("batch", "seq"))
def _embed(ids_t, table, batch, seq):
    mesh = plsc.VectorSubcoreMesh(
        core_axis_name="c", subcore_axis_name="s", num_cores=NC,
        num_subcores=NS)
    out5 = pl.kernel(
        functools.partial(_emb_body, seq),
        out_type=jax.ShapeDtypeStruct(
            (seq, DIM // 8, batch // BBLK, 8, BBLK), jnp.float32),
        mesh=mesh,
        scratch_types=[
            pltpu.VMEM((seq, BBLK), jnp.int32),
            pltpu.VMEM((BBLK, ROWPAD), jnp.float32),
            pltpu.VMEM((BBLK, ROWPAD), jnp.float32),
            pltpu.VMEM((DIM * PITCH + LANES,), jnp.float32),
            pltpu.VMEM((DIM * PITCH + LANES,), jnp.float32),
            pltpu.VMEM((DIM // 8, 8, BBLK), jnp.float32),
            pltpu.VMEM((DIM // 8, 8, BBLK), jnp.float32),
            pltpu.SemaphoreType.DMA,
            pltpu.SemaphoreType.DMA,
            pltpu.SemaphoreType.DMA,
            pltpu.SemaphoreType.DMA,
        ],
        compiler_params=pltpu.CompilerParams(
            use_tc_tiling_on_sc=False, needs_layout_passes=False),
    )(ids_t, table)
    return out5.transpose(2, 4, 0, 1, 3).reshape(batch, seq, DIM)


def kernel(input_ids, table):
    batch, seq = input_ids.shape
    ids_t = input_ids.T.astype(jnp.int32)
    return _embed(ids_t, table, batch, seq)


# speedup vs baseline: 2.4359x; 1.8744x over previous
import functools
import jax
import jax.numpy as jnp
from jax import lax
from jax.experimental import pallas as pl
from jax.experimental.pallas import tpu as pltpu
from jax.experimental.pallas import tpu_sc as plsc

DIM = 64
SCALE = 8.0
LANES = 16

NC = 2
NS = 16
NW = NC * NS

BBLK = 128
ROWPAD = 64
PITCH = 129
NBUF = 2


def _emb_body(seq, ids_hbm, table_hbm, out_hbm, idx_v, g0, g1, f0, f1,
              t0, t1, gs0, gs1, ss0, ss1):
    c = lax.axis_index("c")
    s_ax = lax.axis_index("s")
    wid = s_ax * NC + c

    gbufs, fbufs, tbufs = (g0, g1), (f0, f1), (t0, t1)
    gsems, ssems = (gs0, gs1), (ss0, ss1)

    pltpu.sync_copy(ids_hbm.at[:, pl.ds(wid * BBLK, BBLK)], idx_v)

    for b in range(NBUF):
        pltpu.async_copy(table_hbm.at[idx_v.at[b]], gbufs[b], gsems[b])

    qbase = lax.iota(jnp.int32, LANES)
    qpitch = qbase * PITCH
    ngroups = seq // NBUF

    def group_body(go, carry):
        for b in range(NBUF):
            s = go * NBUF + b
            gbuf, fbuf, tbuf = gbufs[b], fbufs[b], tbufs[b]

            pltpu.make_async_copy(table_hbm.at[idx_v.at[b]], gbuf,
                                  gsems[b]).wait()

            @pl.when(go > 0)
            def _():
                pltpu.make_async_copy(tbuf, out_hbm.at[0, :, 0],
                                      ssems[b]).wait()

            @plsc.parallel_loop(0, BBLK, unroll=4)
            def q_body(q):
                for k in range(DIM // LANES):
                    v = gbuf[q, pl.ds(k * LANES, LANES)]
                    plsc.store_scatter(
                        fbuf, [qpitch + (k * LANES * PITCH + q)], v * SCALE)

            @plsc.parallel_loop(0, DIM, unroll=2)
            def d_body(d):
                dbase = qbase + d * PITCH
                for m in range(BBLK // LANES):
                    v = plsc.load_gather(fbuf, [dbase + m * LANES])
                    tbuf[d // 8, d % 8, pl.ds(m * LANES, LANES)] = v

            pltpu.async_copy(tbuf, out_hbm.at[s, :, wid], ssems[b])

            @pl.when(go < ngroups - 1)
            def _():
                pltpu.async_copy(table_hbm.at[idx_v.at[s + NBUF]], gbuf,
                                 gsems[b])

        return carry

    lax.fori_loop(0, ngroups, group_body, 0)

    for b in range(NBUF):
        pltpu.make_async_copy(tbufs[b], out_hbm.at[0, :, 0], ssems[b]).wait()


@functools.partial(jax.jit, static_argnames=("batch", "seq"))
def _embed(ids_t, table, batch, seq):
    mesh = plsc.VectorSubcoreMesh(
        core_axis_name="c", subcore_axis_name="s", num_cores=NC,
        num_subcores=NS)
    out5 = pl.kernel(
        functools.partial(_emb_body, seq),
        out_type=jax.ShapeDtypeStruct(
            (seq, DIM // 8, batch // BBLK, 8, BBLK), jnp.float32),
        mesh=mesh,
        scratch_types=[
            pltpu.VMEM((seq, BBLK), jnp.int32),
            pltpu.VMEM((BBLK, ROWPAD), jnp.float32),
            pltpu.VMEM((BBLK, ROWPAD), jnp.float32),
            pltpu.VMEM((DIM * PITCH + LANES,), jnp.float32),
            pltpu.VMEM((DIM * PITCH + LANES,), jnp.float32),
            pltpu.VMEM((DIM // 8, 8, BBLK), jnp.float32),
            pltpu.VMEM((DIM // 8, 8, BBLK), jnp.float32),
            pltpu.SemaphoreType.DMA,
            pltpu.SemaphoreType.DMA,
            pltpu.SemaphoreType.DMA,
            pltpu.SemaphoreType.DMA,
        ],
        compiler_params=pltpu.CompilerParams(
            use_tc_tiling_on_sc=False, needs_layout_passes=False),
    )(ids_t, table)
    return out5.transpose(2, 4, 0, 1, 3).reshape(batch, seq, DIM)


def kernel(input_ids, table):
    batch, seq = input_ids.shape
    ids_t = input_ids.T.astype(jnp.int32)
    return _embed(ids_t, table, batch, seq)


# speedup vs baseline: 3.8384x; 1.5757x over previous
import functools
import jax
import jax.numpy as jnp
from jax import lax
from jax.experimental import pallas as pl
from jax.experimental.pallas import tpu as pltpu
from jax.experimental.pallas import tpu_sc as plsc

DIM = 64
SCALE = 8.0
LANES = 16

NC = 2
NS = 16
NW = NC * NS

BBLK = 128
ROWPAD = 64
PITCH = 129
NBUF = 2


def _emb_body(seq, ids_hbm, table_hbm, out_hbm, idx_v, g0, g1, f0, f1,
              t0, t1, gs0, gs1, ss0, ss1):
    c = lax.axis_index("c")
    s_ax = lax.axis_index("s")
    wid = s_ax * NC + c

    gbufs, fbufs, tbufs = (g0, g1), (f0, f1), (t0, t1)
    gsems, ssems = (gs0, gs1), (ss0, ss1)

    pltpu.sync_copy(ids_hbm.at[:, pl.ds(wid * BBLK, BBLK)], idx_v)

    for b in range(NBUF):
        pltpu.async_copy(table_hbm.at[idx_v.at[b]], gbufs[b], gsems[b])

    qbase = lax.iota(jnp.int32, LANES)
    qpitch = qbase * PITCH
    ngroups = seq // NBUF

    def group_body(go, carry):
        for b in range(NBUF):
            s = go * NBUF + b
            gbuf, fbuf, tbuf = gbufs[b], fbufs[b], tbufs[b]

            pltpu.make_async_copy(table_hbm.at[idx_v.at[b]], gbuf,
                                  gsems[b]).wait()

            @pl.when(go > 0)
            def _():
                pltpu.make_async_copy(tbuf, out_hbm.at[0, :, 0],
                                      ssems[b]).wait()

            @plsc.parallel_loop(0, BBLK, unroll=4)
            def q_body(q):
                for k in range(DIM // LANES):
                    v = gbuf[q, pl.ds(k * LANES, LANES)]
                    plsc.store_scatter(
                        fbuf, [qpitch + (k * LANES * PITCH + q)], v * SCALE)

            @plsc.parallel_loop(0, DIM, unroll=2)
            def d_body(d):
                dbase = qbase + d * PITCH
                for m in range(BBLK // LANES):
                    v = plsc.load_gather(fbuf, [dbase + m * LANES])
                    tbuf[d // 8, d % 8, pl.ds(m * LANES, LANES)] = v

            pltpu.async_copy(tbuf, out_hbm.at[s, :, wid], ssems[b])

            @pl.when(go < ngroups - 1)
            def _():
                pltpu.async_copy(table_hbm.at[idx_v.at[s + NBUF]], gbuf,
                                 gsems[b])

        return carry

    lax.fori_loop(0, ngroups, group_body, 0)

    for b in range(NBUF):
        pltpu.make_async_copy(tbufs[b], out_hbm.at[0, :, 0], ssems[b]).wait()


def _tr_block(src, fbuf, dst, nq, qbase):
    @plsc.parallel_loop(0, DIM, unroll=4)
    def d_body(d):
        for m in range(nq // LANES):
            v = src[d, pl.ds(m * LANES, LANES)]
            plsc.store_scatter(
                fbuf, [(qbase + m * LANES) * PITCH + d], v)

    @plsc.parallel_loop(0, nq, unroll=4)
    def q_body(q):
        for m in range(DIM // LANES):
            v = plsc.load_gather(fbuf, [qbase + (q * PITCH + m * LANES)])
            dst[q, pl.ds(m * LANES, LANES)] = v


def _tr_body(nfull, tableT_hbm, tail_hbm, t128_hbm, i0, i1, f0, f1, o0, o1,
             gs0, gs1, ss0, ss1):
    c = lax.axis_index("c")
    s_ax = lax.axis_index("s")
    wid = s_ax * NC + c

    ibufs, fbufs, obufs = (i0, i1), (f0, f1), (o0, o1)
    gsems, ssems = (gs0, gs1), (ss0, ss1)
    qbase = lax.iota(jnp.int32, LANES)
    nv = tableT_hbm.shape[1]

    def blk(g):
        return jnp.minimum(wid + g * NW, nfull - 1) * BBLK

    for b in range(NBUF):
        pltpu.async_copy(tableT_hbm.at[:, pl.ds(blk(b), BBLK)], ibufs[b],
                         gsems[b])

    ngroups = (nfull + NW - 1) // NW
    npipe = ((ngroups + NBUF - 1) // NBUF) * NBUF

    def group_body(go, carry):
        for b in range(NBUF):
            g = go * NBUF + b
            ibuf, fbuf, obuf = ibufs[b], fbufs[b], obufs[b]
            pltpu.make_async_copy(tableT_hbm.at[:, pl.ds(0, BBLK)], ibuf,
                                  gsems[b]).wait()

            @pl.when(go > 0)
            def _():
                pltpu.make_async_copy(obuf, t128_hbm.at[pl.ds(0, BBLK)],
                                      ssems[b]).wait()

            _tr_block(ibuf, fbuf, obuf, BBLK, qbase)

            pltpu.async_copy(obuf, t128_hbm.at[pl.ds(blk(g), BBLK)],
                             ssems[b])

            @pl.when(g + NBUF < npipe)
            def _():
                pltpu.async_copy(tableT_hbm.at[:, pl.ds(blk(g + NBUF), BBLK)],
                                 ibuf, gsems[b])

        return carry

    lax.fori_loop(0, npipe // NBUF, group_body, 0)

    for b in range(NBUF):
        pltpu.make_async_copy(obufs[b], t128_hbm.at[pl.ds(0, BBLK)],
                              ssems[b]).wait()

    @pl.when(wid == 0)
    def _():
        pltpu.sync_copy(tail_hbm, i0)
        _tr_block(i0, f0, o0, BBLK, qbase)
        pltpu.sync_copy(o0, t128_hbm.at[pl.ds(nv - BBLK, BBLK)])


@functools.partial(jax.jit, static_argnames=("batch", "seq"))
def _embed(ids_t, table_t, batch, seq):
    vocab = table_t.shape[1]
    mesh = plsc.VectorSubcoreMesh(
        core_axis_name="c", subcore_axis_name="s", num_cores=NC,
        num_subcores=NS)
    nfull = vocab // BBLK
    t128 = pl.kernel(
        functools.partial(_tr_body, nfull),
        out_type=jax.ShapeDtypeStruct((vocab, 2 * DIM), jnp.float32),
        mesh=mesh,
        scratch_types=[
            pltpu.VMEM((DIM, BBLK), jnp.float32),
            pltpu.VMEM((DIM, BBLK), jnp.float32),
            pltpu.VMEM((BBLK * PITCH + LANES,), jnp.float32),
            pltpu.VMEM((BBLK * PITCH + LANES,), jnp.float32),
            pltpu.VMEM((BBLK, 2 * DIM), jnp.float32),
            pltpu.VMEM((BBLK, 2 * DIM), jnp.float32),
            pltpu.SemaphoreType.DMA,
            pltpu.SemaphoreType.DMA,
            pltpu.SemaphoreType.DMA,
            pltpu.SemaphoreType.DMA,
        ],
        compiler_params=pltpu.CompilerParams(
            use_tc_tiling_on_sc=True, needs_layout_passes=False),
    )(table_t, table_t[:, vocab - BBLK:])
    table2 = t128.reshape(2 * vocab, DIM)
    return _embed2(ids_t * 2, table2, batch, seq)


def _embed2(ids_t, table, batch, seq):
    mesh = plsc.VectorSubcoreMesh(
        core_axis_name="c", subcore_axis_name="s", num_cores=NC,
        num_subcores=NS)
    out5 = pl.kernel(
        functools.partial(_emb_body, seq),
        out_type=jax.ShapeDtypeStruct(
            (seq, DIM // 8, batch // BBLK, 8, BBLK), jnp.float32),
        mesh=mesh,
        scratch_types=[
            pltpu.VMEM((seq, BBLK), jnp.int32),
            pltpu.VMEM((BBLK, ROWPAD), jnp.float32),
            pltpu.VMEM((BBLK, ROWPAD), jnp.float32),
            pltpu.VMEM((DIM * PITCH + LANES,), jnp.float32),
            pltpu.VMEM((DIM * PITCH + LANES,), jnp.float32),
            pltpu.VMEM((DIM // 8, 8, BBLK), jnp.float32),
            pltpu.VMEM((DIM // 8, 8, BBLK), jnp.float32),
            pltpu.SemaphoreType.DMA,
            pltpu.SemaphoreType.DMA,
            pltpu.SemaphoreType.DMA,
            pltpu.SemaphoreType.DMA,
        ],
        compiler_params=pltpu.CompilerParams(
            use_tc_tiling_on_sc=False, needs_layout_passes=False),
    )(ids_t, table)
    return out5.transpose(2, 4, 0, 1, 3).reshape(batch, seq, DIM)


def kernel(input_ids, table):
    batch, seq = input_ids.shape
    ids_t = input_ids.T.astype(jnp.int32)
    return _embed(ids_t, table.T, batch, seq)


# speedup vs baseline: 4.2297x; 1.1019x over previous
import functools
import jax
import jax.numpy as jnp
from jax import lax
from jax.experimental import pallas as pl
from jax.experimental.pallas import tpu as pltpu
from jax.experimental.pallas import tpu_sc as plsc

DIM = 64
SCALE = 8.0
LANES = 16

NC = 2
NS = 16
NW = NC * NS

BBLK = 128
ROWPAD = 64
PITCH = 129
NBUF = 2


def _emb_body(seq, ids_hbm, table_hbm, out_hbm, idx_v, g0, g1, f0, f1,
              t0, t1, gs0, gs1, ss0, ss1):
    c = lax.axis_index("c")
    s_ax = lax.axis_index("s")
    wid = s_ax * NC + c

    gbufs, fbufs, tbufs = (g0, g1), (f0, f1), (t0, t1)
    gsems, ssems = (gs0, gs1), (ss0, ss1)

    pltpu.sync_copy(ids_hbm.at[:, pl.ds(wid * BBLK, BBLK)], idx_v)

    for b in range(NBUF):
        pltpu.async_copy(table_hbm.at[idx_v.at[b]], gbufs[b], gsems[b])

    qbase = lax.iota(jnp.int32, LANES)
    qpitch = qbase * PITCH
    ngroups = seq // NBUF

    def group_body(go, carry):
        for b in range(NBUF):
            s = go * NBUF + b
            gbuf, fbuf, tbuf = gbufs[b], fbufs[b], tbufs[b]

            pltpu.make_async_copy(table_hbm.at[idx_v.at[b]], gbuf,
                                  gsems[b]).wait()

            @pl.when(go > 0)
            def _():
                pltpu.make_async_copy(tbuf, out_hbm.at[0, :, 0],
                                      ssems[b]).wait()

            @plsc.parallel_loop(0, BBLK, unroll=4)
            def q_body(q):
                for k in range(DIM // LANES):
                    v = gbuf[q, pl.ds(k * LANES, LANES)]
                    plsc.store_scatter(
                        fbuf, [qpitch + (k * LANES * PITCH + q)], v * SCALE)

            @plsc.parallel_loop(0, DIM, unroll=2)
            def d_body(d):
                dbase = qbase + d * PITCH
                for m in range(BBLK // LANES):
                    v = plsc.load_gather(fbuf, [dbase + m * LANES])
                    tbuf[d // 8, d % 8, pl.ds(m * LANES, LANES)] = v

            pltpu.async_copy(tbuf, out_hbm.at[s, :, wid], ssems[b])

            @pl.when(go < ngroups - 1)
            def _():
                pltpu.async_copy(table_hbm.at[idx_v.at[s + NBUF]], gbuf,
                                 gsems[b])

        return carry

    lax.fori_loop(0, ngroups, group_body, 0)

    for b in range(NBUF):
        pltpu.make_async_copy(tbufs[b], out_hbm.at[0, :, 0], ssems[b]).wait()


def _tr_block(src, fbuf, dst, nq, qbase):
    @plsc.parallel_loop(0, DIM, unroll=4)
    def d_body(d):
        for m in range(nq // LANES):
            v = src[d, pl.ds(m * LANES, LANES)]
            plsc.store_scatter(
                fbuf, [(qbase + m * LANES) * PITCH + d], v)

    @plsc.parallel_loop(0, nq, unroll=4)
    def q_body(q):
        col = (q % 2) * DIM
        for m in range(DIM // LANES):
            v = plsc.load_gather(fbuf, [qbase + (q * PITCH + m * LANES)])
            dst[q // 2, pl.ds(col + m * LANES, LANES)] = v


def _tr_body(nfull, tableT_hbm, tail_hbm, t128_hbm, i0, i1, f0, f1, o0, o1,
             gs0, gs1, ss0, ss1):
    c = lax.axis_index("c")
    s_ax = lax.axis_index("s")
    wid = s_ax * NC + c

    ibufs, fbufs, obufs = (i0, i1), (f0, f1), (o0, o1)
    gsems, ssems = (gs0, gs1), (ss0, ss1)
    qbase = lax.iota(jnp.int32, LANES)
    nv = tableT_hbm.shape[1]

    def blk(g):
        return jnp.minimum(wid + g * NW, nfull - 1) * (BBLK // 2)

    for b in range(NBUF):
        pltpu.async_copy(tableT_hbm.at[:, pl.ds(blk(b) * 2, BBLK)], ibufs[b],
                         gsems[b])

    ngroups = (nfull + NW - 1) // NW
    npipe = ((ngroups + NBUF - 1) // NBUF) * NBUF

    def group_body(go, carry):
        for b in range(NBUF):
            g = go * NBUF + b
            ibuf, fbuf, obuf = ibufs[b], fbufs[b], obufs[b]
            pltpu.make_async_copy(tableT_hbm.at[:, pl.ds(0, BBLK)], ibuf,
                                  gsems[b]).wait()

            @pl.when(go > 0)
            def _():
                pltpu.make_async_copy(obuf, t128_hbm.at[pl.ds(0, BBLK // 2)],
                                      ssems[b]).wait()

            _tr_block(ibuf, fbuf, obuf, BBLK, qbase)

            pltpu.async_copy(obuf, t128_hbm.at[pl.ds(blk(g), BBLK // 2)],
                             ssems[b])

            @pl.when(g + NBUF < npipe)
            def _():
                pltpu.async_copy(
                    tableT_hbm.at[:, pl.ds(blk(g + NBUF) * 2, BBLK)],
                    ibuf, gsems[b])

        return carry

    lax.fori_loop(0, npipe // NBUF, group_body, 0)

    for b in range(NBUF):
        pltpu.make_async_copy(obufs[b], t128_hbm.at[pl.ds(0, BBLK // 2)],
                              ssems[b]).wait()

    @pl.when(wid == 0)
    def _():
        pltpu.sync_copy(tail_hbm, i0)
        _tr_block(i0, f0, o0, BBLK, qbase)
        pltpu.sync_copy(o0, t128_hbm.at[pl.ds((nv - BBLK) // 2, BBLK // 2)])


@functools.partial(jax.jit, static_argnames=("batch", "seq"))
def _embed(ids_t, table_t, batch, seq):
    vocab = table_t.shape[1]
    mesh = plsc.VectorSubcoreMesh(
        core_axis_name="c", subcore_axis_name="s", num_cores=NC,
        num_subcores=NS)
    nfull = vocab // BBLK
    t128 = pl.kernel(
        functools.partial(_tr_body, nfull),
        out_type=jax.ShapeDtypeStruct((vocab // 2, 2 * DIM), jnp.float32),
        mesh=mesh,
        scratch_types=[
            pltpu.VMEM((DIM, BBLK), jnp.float32),
            pltpu.VMEM((DIM, BBLK), jnp.float32),
            pltpu.VMEM((BBLK * PITCH + LANES,), jnp.float32),
            pltpu.VMEM((BBLK * PITCH + LANES,), jnp.float32),
            pltpu.VMEM((BBLK // 2, 2 * DIM), jnp.float32),
            pltpu.VMEM((BBLK // 2, 2 * DIM), jnp.float32),
            pltpu.SemaphoreType.DMA,
            pltpu.SemaphoreType.DMA,
            pltpu.SemaphoreType.DMA,
            pltpu.SemaphoreType.DMA,
        ],
        compiler_params=pltpu.CompilerParams(
            use_tc_tiling_on_sc=True, needs_layout_passes=False),
    )(table_t, table_t[:, vocab - BBLK:])
    table2 = t128.reshape(vocab, DIM)
    return _embed2(ids_t, table2, batch, seq)


def _embed2(ids_t, table, batch, seq):
    mesh = plsc.VectorSubcoreMesh(
        core_axis_name="c", subcore_axis_name="s", num_cores=NC,
        num_subcores=NS)
    out5 = pl.kernel(
        functools.partial(_emb_body, seq),
        out_type=jax.ShapeDtypeStruct(
            (seq, DIM // 8, batch // BBLK, 8, BBLK), jnp.float32),
        mesh=mesh,
        scratch_types=[
            pltpu.VMEM((seq, BBLK), jnp.int32),
            pltpu.VMEM((BBLK, ROWPAD), jnp.float32),
            pltpu.VMEM((BBLK, ROWPAD), jnp.float32),
            pltpu.VMEM((DIM * PITCH + LANES,), jnp.float32),
            pltpu.VMEM((DIM * PITCH + LANES,), jnp.float32),
            pltpu.VMEM((DIM // 8, 8, BBLK), jnp.float32),
            pltpu.VMEM((DIM // 8, 8, BBLK), jnp.float32),
            pltpu.SemaphoreType.DMA,
            pltpu.SemaphoreType.DMA,
            pltpu.SemaphoreType.DMA,
            pltpu.SemaphoreType.DMA,
        ],
        compiler_params=pltpu.CompilerParams(
            use_tc_tiling_on_sc=False, needs_layout_passes=False),
    )(ids_t, table)
    return out5.transpose(2, 4, 0, 1, 3).reshape(batch, seq, DIM)


def kernel(input_ids, table):
    batch, seq = input_ids.shape
    ids_t = input_ids.T.astype(jnp.int32)
    return _embed(ids_t, table.T, batch, seq)
